# baseline (speedup 1.0000x reference)
import functools

import jax
import jax.numpy as jnp
from jax import lax
from jax.experimental import pallas as pl
from jax.experimental.pallas import tpu as pltpu
from jax.experimental.pallas import tpu_sc as plsc

B, D = 2048, 2048
O, F = 2048, 32

NC, NS = 2, 16
NW = NC * NS
OH = O // 2
RPW = OH // NW
OBLK = 16
NBLK = RPW // OBLK
_ZUNROLL = 8


def _make_sc_body(base):
    def body(idx_hbm, w_hbm, out_hbm, buf0, buf1, idxs, ws, sem0, sem1):
        wid = lax.axis_index("s") * NC + lax.axis_index("c")
        zeros16 = jnp.zeros((16,), jnp.float32)
        r0 = wid * RPW

        pltpu.sync_copy(idx_hbm.at[pl.ds(base + r0, RPW)], idxs)
        pltpu.sync_copy(w_hbm.at[pl.ds(base + r0, RPW)], ws)

        for buf in (buf0, buf1):
            for r in range(OBLK):

                def zero_body(i, carry, buf=buf, r=r):
                    c = i * 16 * _ZUNROLL
                    for u in range(_ZUNROLL):
                        buf[r, pl.ds(c + u * 16, 16)] = zeros16
                    return carry

                lax.fori_loop(0, D // (16 * _ZUNROLL), zero_body, 0)

        bufs = (buf0, buf1)
        sems = (sem0, sem1)
        copies = []
        for t in range(NBLK):
            buf = bufs[t]
            for ol in range(OBLK):
                cur = t * OBLK + ol
                row = jnp.full((16,), ol, jnp.int32)
                for h in range(F // 16):
                    iv = idxs[cur, pl.ds(h * 16, 16)]
                    wv = ws[cur, pl.ds(h * 16, 16)]
                    plsc.addupdate_scatter(buf, [row, iv], wv)
            cp = pltpu.make_async_copy(
                buf, out_hbm.at[pl.ds(r0 + t * OBLK, OBLK)], sems[t]
            )
            cp.start()
            copies.append(cp)
        for cp in copies:
            cp.wait()

    return body


@functools.cache
def _sc_scatter_half(base):
    return functools.partial(
        pl.kernel,
        out_type=jax.ShapeDtypeStruct((OH, D), jnp.float32),
        mesh=plsc.VectorSubcoreMesh(
            core_axis_name="c", subcore_axis_name="s", num_cores=NC, num_subcores=NS
        ),
        scratch_types=[
            pltpu.VMEM((OBLK, D), jnp.float32),
            pltpu.VMEM((OBLK, D), jnp.float32),
            pltpu.VMEM((RPW, F), jnp.int32),
            pltpu.VMEM((RPW, F), jnp.float32),
            pltpu.SemaphoreType.DMA,
            pltpu.SemaphoreType.DMA,
        ],
        compiler_params=pltpu.CompilerParams(
            use_tc_tiling_on_sc=True, needs_layout_passes=False
        ),
    )(_make_sc_body(base))


TB = 256


def _mm_body0(x_ref, w_ref, b_ref, o_ref):
    o_ref[...] = (
        lax.dot_general(
            x_ref[...],
            w_ref[...],
            (((1,), (1,)), ((), ())),
            preferred_element_type=jnp.float32,
        )
        + b_ref[...]
    )


def _mm_body1(x_ref, w_ref, b_ref, prev_ref, o_ref):
    del prev_ref
    _mm_body0(x_ref, w_ref, b_ref, o_ref)


_mm_half0 = pl.pallas_call(
    _mm_body0,
    grid=(B // TB,),
    in_specs=[
        pl.BlockSpec((TB, D), lambda i: (i, 0)),
        pl.BlockSpec((OH, D), lambda i: (0, 0)),
        pl.BlockSpec((1, OH), lambda i: (0, 0)),
    ],
    out_specs=pl.BlockSpec((TB, OH), lambda i: (i, 0)),
    out_shape=jax.ShapeDtypeStruct((B, O), jnp.float32),
)

_mm_half1 = pl.pallas_call(
    _mm_body1,
    grid=(B // TB,),
    in_specs=[
        pl.BlockSpec((TB, D), lambda i: (i, 0)),
        pl.BlockSpec((OH, D), lambda i: (0, 0)),
        pl.BlockSpec((1, OH), lambda i: (0, 0)),
        pl.BlockSpec(memory_space=pl.ANY),
    ],
    out_specs=pl.BlockSpec((TB, OH), lambda i: (i, 1)),
    out_shape=jax.ShapeDtypeStruct((B, O), jnp.float32),
    input_output_aliases={3: 0},
)


def kernel(input, indx_seqs, weight, bias):
    idx32 = indx_seqs.astype(jnp.int32)
    w1 = _sc_scatter_half(0)(idx32, weight)
    w2 = _sc_scatter_half(OH)(idx32, weight)
    b2d = bias.reshape(1, O)
    out = _mm_half0(input, w1, b2d[:, :OH])
    out = _mm_half1(input, w2, b2d[:, OH:], out)
    return out
